# baseline (speedup 1.0000x reference)
import functools
import jax
import jax.numpy as jnp
from jax import lax
from jax.experimental import pallas as pl
from jax.experimental.pallas import tpu as pltpu
from jax.experimental.pallas import tpu_sc as plsc

K = 64
B = 64
N = 32768
NC = 2
NS = 16
NW = NC * NS
ROWS_PER_W = B // NW
NEG_INF = float("-inf")


def _lane_shuffle(x, k):
    idx = lax.iota(jnp.int32, 16) ^ jnp.int32(k)
    return x.at[idx].get(mode="promise_in_bounds")


def _all_max(x):
    for k in (8, 4, 2, 1):
        x = jnp.maximum(x, _lane_shuffle(x, k))
    return x


def _all_sum(x):
    for k in (8, 4, 2, 1):
        x = x + _lane_shuffle(x, k)
    return x


def _untransform_splat(key_u32_scalar):
    t = jnp.full((16,), key_u32_scalar, jnp.uint32)
    hi = t >= jnp.uint32(0x80000000)
    bits = jnp.where(hi, t ^ jnp.uint32(0x80000000), ~t)
    return lax.bitcast_convert_type(bits, jnp.float32)


def _transform_key(v):
    ui = lax.bitcast_convert_type(v, jnp.uint32)
    neg = ui >= jnp.uint32(0x80000000)
    return jnp.where(neg, ~ui, ui | jnp.uint32(0x80000000))


def _insert2(r1, r2, v):
    lo = jnp.minimum(r1, v)
    return jnp.maximum(r1, v), jnp.maximum(r2, lo)


U = 8
CH = N // 4


def _process_row(row_ref, fbuf, seq_len, chunk_waits):
    iota = lax.iota(jnp.int32, 16)
    nfull = seq_len // 16
    ntail = seq_len - nfull * 16
    ng = nfull // U

    def p1_body(g, carry):
        a1, a2, b1, b2 = carry
        base = g * (U * 16)
        vs = [row_ref[pl.ds(base + k * 16, 16)] for k in range(U)]
        for k in range(0, U, 2):
            a1, a2 = _insert2(a1, a2, vs[k])
        for k in range(1, U, 2):
            b1, b2 = _insert2(b1, b2, vs[k])
        return (a1, a2, b1, b2)

    ninf = jnp.full((16,), NEG_INF, jnp.float32)
    carry = (ninf, ninf, ninf, ninf)
    if chunk_waits is None:
        carry = lax.fori_loop(0, ng, p1_body, carry)
    else:
        gpc = CH // (16 * U)
        for ci, wait in enumerate(chunk_waits):
            wait()
            lo = jnp.minimum(ng, ci * gpc)
            hi = jnp.minimum(ng, (ci + 1) * gpc)
            carry = lax.fori_loop(lo, hi, p1_body, carry)
    a1, a2, b1, b2 = carry

    def p1r_body(i, carry):
        a1, a2, b1, b2 = carry
        v = row_ref[pl.ds(i * 16, 16)]
        a1, a2 = _insert2(a1, a2, v)
        return (a1, a2, b1, b2)

    a1, a2, b1, b2 = lax.fori_loop(ng * U, nfull, p1r_body, (a1, a2, b1, b2))
    vt = row_ref[pl.ds(nfull * 16, 16)]
    vt = jnp.where(iota < ntail, vt, NEG_INF)
    b1, b2 = _insert2(b1, b2, vt)
    tlb_vec = -_all_max(-jnp.minimum(a2, b2))
    vmax_vec = _all_max(jnp.maximum(a1, b1))
    t_lb = tlb_vec[0]

    kl = _transform_key(tlb_vec)[0]
    km = _transform_key(vmax_vec)[0]

    neg = jnp.full((16,), NEG_INF, jnp.float32)

    def p2_store(c, v, amax):
        fbuf[pl.ds(c, 16)] = v
        return jnp.where(amax[0] > t_lb, c + 16, c)

    def p2_body(g, c):
        base = g * (U * 16)
        vs, amaxs = [], []
        for k in range(U):
            v = row_ref[pl.ds(base + k * 16, 16)]
            vs.append(v)
            amaxs.append(_all_max(v))
        for k in range(U):
            c = p2_store(c, vs[k], amaxs[k])
        return c

    c = lax.fori_loop(0, ng, p2_body, jnp.int32(0))

    def p2r_body(i, c):
        v = row_ref[pl.ds(i * 16, 16)]
        return p2_store(c, v, _all_max(v))

    c = lax.fori_loop(ng * U, nfull, p2r_body, c)
    vt2 = row_ref[pl.ds(nfull * 16, 16)]
    mt = jnp.where(iota < ntail, vt2, NEG_INF)
    c = p2_store(c, mt, _all_max(mt))

    for j in range(7):
        fbuf[pl.ds(c + j * 16, 16)] = neg
    c_r = jnp.bitwise_and(c + 127, jnp.int32(-128))
    nv8 = c_r >> 7
    k_i = jnp.minimum(seq_len, K)

    def scan8(vcand, strict):
        def body(i, accs):
            base = i * 128
            out = list(accs)
            for j in range(8):
                v = fbuf[pl.ds(base + j * 16, 16)]
                hit = (v > vcand) if strict else (v >= vcand)
                out[j & 3] = out[j & 3] + jnp.where(hit, 1, 0).astype(
                    jnp.int32)
            return tuple(out)
        zero = jnp.zeros((16,), jnp.int32)
        accs = lax.fori_loop(0, nv8, body, (zero,) * 4)
        return _all_sum((accs[0] + accs[1]) + (accs[2] + accs[3]))[0]

    tlb_splat = jnp.full((16,), t_lb, jnp.float32)
    csurv = scan8(tlb_splat, True)
    sentinel = csurv < k_i

    diff = kl ^ km
    for s in (1, 2, 4, 8, 16):
        diff = diff | (diff >> s)
    t0 = km & ~diff

    dlow = jnp.bitwise_and(diff, jnp.uint32(0x7FFFFFFF)).astype(jnp.int32)
    fexp = lax.bitcast_convert_type(
        dlow.astype(jnp.float32), jnp.int32) >> 23
    nbits = jnp.where(diff >= jnp.uint32(0x80000000),
                      jnp.int32(32), fexp - 126)
    nbits = jnp.where(sentinel, jnp.int32(0), nbits)

    def radix_body(i, carry):
        d, t = carry
        cand = t | (d ^ (d >> 1))
        cnt = scan8(_untransform_splat(cand), False)
        return (d >> 1, jnp.where(cnt >= k_i, cand, t))

    _, t = lax.fori_loop(0, nbits, radix_body, (diff, t0))

    tval_splat = jnp.where(sentinel, tlb_splat, _untransform_splat(t))

    def sum_body(i, carry):
        svs, cvs = carry
        base = i * 128
        svo, cvo = list(svs), list(cvs)
        for j in range(8):
            v = fbuf[pl.ds(base + j * 16, 16)]
            m = v > tval_splat
            svo[j & 3] = svo[j & 3] + jnp.where(m, v, jnp.float32(0.0))
            cvo[j & 3] = cvo[j & 3] + jnp.where(m, 1, 0).astype(jnp.int32)
        return (tuple(svo), tuple(cvo))

    zf = jnp.zeros((16,), jnp.float32)
    zi = jnp.zeros((16,), jnp.int32)
    svs, cvs = lax.fori_loop(0, nv8, sum_body, ((zf,) * 4, (zi,) * 4))
    sum_gt_vec = _all_sum((svs[0] + svs[1]) + (svs[2] + svs[3]))
    cnt_gt_vec = _all_sum((cvs[0] + cvs[1]) + (cvs[2] + cvs[3]))

    ki_vec = jnp.full((16,), k_i, jnp.int32)
    tie_vec = ki_vec - cnt_gt_vec
    total_vec = sum_gt_vec + jnp.where(
        tie_vec > 0, tie_vec.astype(jnp.float32) * tval_splat,
        jnp.float32(0.0))
    return total_vec / ki_vec.astype(jnp.float32)


def _sc_body(scores_hbm, seqlen_hbm, out_hbm,
             rows_v, fbuf, seql_v, outv, sem0a, sem0b, sem0c, sem0d, sem1):
    w = lax.axis_index("s") * NC + lax.axis_index("c")
    r0 = w * ROWS_PER_W
    cps = [
        pltpu.async_copy(scores_hbm.at[r0, pl.ds(j * CH, CH)],
                         rows_v.at[0, pl.ds(j * CH, CH)], sem)
        for j, sem in enumerate((sem0a, sem0b, sem0c, sem0d))
    ]
    cp1 = pltpu.async_copy(scores_hbm.at[r0 + 1], rows_v.at[1, pl.ds(0, N)],
                           sem1)
    pltpu.sync_copy(seqlen_hbm, seql_v.at[pl.ds(0, B)])

    seq_vec = seql_v[pl.ds(r0, 16)]
    mean0 = _process_row(rows_v.at[0], fbuf, seq_vec[0],
                         [cp.wait for cp in cps])
    cp1.wait()
    mean1 = _process_row(rows_v.at[1], fbuf, seq_vec[1], None)

    iota = lax.iota(jnp.int32, 16)
    vec = jnp.where(iota == 0, mean0,
                    jnp.where(iota == 1, mean1, jnp.float32(0.0)))
    outv[...] = vec.astype(jnp.float32)
    pltpu.sync_copy(outv, out_hbm.at[w])


_sc_means = functools.partial(
    pl.kernel,
    out_type=jax.ShapeDtypeStruct((NW, 16), jnp.float32),
    mesh=plsc.VectorSubcoreMesh(core_axis_name="c", subcore_axis_name="s"),
    scratch_types=[
        pltpu.VMEM((ROWS_PER_W, N + 16), jnp.float32),
        pltpu.VMEM((N + 128,), jnp.float32),
        pltpu.VMEM((B + 16,), jnp.int32),
        pltpu.VMEM((16,), jnp.float32),
        pltpu.SemaphoreType.DMA,
        pltpu.SemaphoreType.DMA,
        pltpu.SemaphoreType.DMA,
        pltpu.SemaphoreType.DMA,
        pltpu.SemaphoreType.DMA,
    ],
)(_sc_body)


def _bce_body(means_ref, label_ref, out_ref):
    m = means_ref[...][:, :ROWS_PER_W]
    lab = label_ref[...]
    p = jax.nn.sigmoid(m)
    eps = 1e-7
    p = jnp.clip(p, eps, 1.0 - eps)
    bce = -(lab * jnp.log(p) + (1.0 - lab) * jnp.log(1.0 - p))
    out_ref[0, 0] = jnp.mean(bce)


@jax.jit
def kernel(scores, label, seqlen):
    means_tile = _sc_means(scores, seqlen.astype(jnp.int32))
    out = pl.pallas_call(
        _bce_body,
        out_shape=jax.ShapeDtypeStruct((1, 1), jnp.float32),
        out_specs=pl.BlockSpec(memory_space=pltpu.SMEM),
    )(means_tile, label.reshape(NW, ROWS_PER_W))
    return out[0, 0]


# speedup vs baseline: 1.0101x; 1.0101x over previous
import functools
import jax
import jax.numpy as jnp
from jax import lax
from jax.experimental import pallas as pl
from jax.experimental.pallas import tpu as pltpu
from jax.experimental.pallas import tpu_sc as plsc

K = 64
B = 64
N = 32768
NC = 2
NS = 16
NW = NC * NS
ROWS_PER_W = B // NW
NEG_INF = float("-inf")


def _lane_shuffle(x, k):
    idx = lax.iota(jnp.int32, 16) ^ jnp.int32(k)
    return x.at[idx].get(mode="promise_in_bounds")


def _all_max(x):
    for k in (8, 4, 2, 1):
        x = jnp.maximum(x, _lane_shuffle(x, k))
    return x


def _all_sum(x):
    for k in (8, 4, 2, 1):
        x = x + _lane_shuffle(x, k)
    return x


def _untransform_splat(key_u32_scalar):
    t = jnp.full((16,), key_u32_scalar, jnp.uint32)
    hi = t >= jnp.uint32(0x80000000)
    bits = jnp.where(hi, t ^ jnp.uint32(0x80000000), ~t)
    return lax.bitcast_convert_type(bits, jnp.float32)


def _transform_key(v):
    ui = lax.bitcast_convert_type(v, jnp.uint32)
    neg = ui >= jnp.uint32(0x80000000)
    return jnp.where(neg, ~ui, ui | jnp.uint32(0x80000000))


def _insert2(r1, r2, v):
    lo = jnp.minimum(r1, v)
    return jnp.maximum(r1, v), jnp.maximum(r2, lo)


U = 8


def _process_row(row_ref, fbuf, seq_len):
    iota = lax.iota(jnp.int32, 16)
    nfull = seq_len // 16
    ntail = seq_len - nfull * 16
    ng = nfull // U

    def p1_body(g, carry):
        a1, a2, b1, b2 = carry
        base = g * (U * 16)
        vs = [row_ref[pl.ds(base + k * 16, 16)] for k in range(U)]
        for k in range(0, U, 2):
            a1, a2 = _insert2(a1, a2, vs[k])
        for k in range(1, U, 2):
            b1, b2 = _insert2(b1, b2, vs[k])
        return (a1, a2, b1, b2)

    ninf = jnp.full((16,), NEG_INF, jnp.float32)
    a1, a2, b1, b2 = lax.fori_loop(0, ng, p1_body, (ninf, ninf, ninf, ninf))

    def p1r_body(i, carry):
        a1, a2, b1, b2 = carry
        v = row_ref[pl.ds(i * 16, 16)]
        a1, a2 = _insert2(a1, a2, v)
        return (a1, a2, b1, b2)

    a1, a2, b1, b2 = lax.fori_loop(ng * U, nfull, p1r_body, (a1, a2, b1, b2))
    vt = row_ref[pl.ds(nfull * 16, 16)]
    vt = jnp.where(iota < ntail, vt, NEG_INF)
    b1, b2 = _insert2(b1, b2, vt)
    tlb_vec = -_all_max(-jnp.minimum(a2, b2))
    vmax_vec = _all_max(jnp.maximum(a1, b1))
    t_lb = tlb_vec[0]

    kl = _transform_key(tlb_vec)[0]
    km = _transform_key(vmax_vec)[0]

    neg = jnp.full((16,), NEG_INF, jnp.float32)

    def p2_store(c, v, amax):
        fbuf[pl.ds(c, 16)] = v
        return jnp.where(amax[0] > t_lb, c + 16, c)

    def p2_body(g, c):
        base = g * (U * 16)
        vs, amaxs = [], []
        for k in range(U):
            v = row_ref[pl.ds(base + k * 16, 16)]
            vs.append(v)
            amaxs.append(_all_max(v))
        for k in range(U):
            c = p2_store(c, vs[k], amaxs[k])
        return c

    c = lax.fori_loop(0, ng, p2_body, jnp.int32(0))

    def p2r_body(i, c):
        v = row_ref[pl.ds(i * 16, 16)]
        return p2_store(c, v, _all_max(v))

    c = lax.fori_loop(ng * U, nfull, p2r_body, c)
    vt2 = row_ref[pl.ds(nfull * 16, 16)]
    mt = jnp.where(iota < ntail, vt2, NEG_INF)
    c = p2_store(c, mt, _all_max(mt))

    for j in range(7):
        fbuf[pl.ds(c + j * 16, 16)] = neg
    c_r = jnp.bitwise_and(c + 127, jnp.int32(-128))
    nv8 = c_r >> 7
    k_i = jnp.minimum(seq_len, K)

    def scan8(vcand, strict):
        def body(i, accs):
            base = i * 128
            out = list(accs)
            for j in range(8):
                v = fbuf[pl.ds(base + j * 16, 16)]
                hit = (v > vcand) if strict else (v >= vcand)
                out[j & 3] = out[j & 3] + jnp.where(hit, 1, 0).astype(
                    jnp.int32)
            return tuple(out)
        zero = jnp.zeros((16,), jnp.int32)
        accs = lax.fori_loop(0, nv8, body, (zero,) * 4)
        return _all_sum((accs[0] + accs[1]) + (accs[2] + accs[3]))[0]

    tlb_splat = jnp.full((16,), t_lb, jnp.float32)
    csurv = scan8(tlb_splat, True)
    sentinel = csurv < k_i

    diff = kl ^ km
    for s in (1, 2, 4, 8, 16):
        diff = diff | (diff >> s)
    t0 = km & ~diff

    dlow = jnp.bitwise_and(diff, jnp.uint32(0x7FFFFFFF)).astype(jnp.int32)
    fexp = lax.bitcast_convert_type(
        dlow.astype(jnp.float32), jnp.int32) >> 23
    nbits = jnp.where(diff >= jnp.uint32(0x80000000),
                      jnp.int32(32), fexp - 126)
    nbits = jnp.where(sentinel, jnp.int32(0), nbits)

    def radix_body(i, carry):
        d, t = carry
        cand = t | (d ^ (d >> 1))
        cnt = scan8(_untransform_splat(cand), False)
        return (d >> 1, jnp.where(cnt >= k_i, cand, t))

    _, t = lax.fori_loop(0, nbits, radix_body, (diff, t0))

    tval_splat = jnp.where(sentinel, tlb_splat, _untransform_splat(t))

    def sum_body(i, carry):
        svs, cvs = carry
        base = i * 128
        svo, cvo = list(svs), list(cvs)
        for j in range(8):
            v = fbuf[pl.ds(base + j * 16, 16)]
            m = v > tval_splat
            svo[j & 3] = svo[j & 3] + jnp.where(m, v, jnp.float32(0.0))
            cvo[j & 3] = cvo[j & 3] + jnp.where(m, 1, 0).astype(jnp.int32)
        return (tuple(svo), tuple(cvo))

    zf = jnp.zeros((16,), jnp.float32)
    zi = jnp.zeros((16,), jnp.int32)
    svs, cvs = lax.fori_loop(0, nv8, sum_body, ((zf,) * 4, (zi,) * 4))
    sum_gt_vec = _all_sum((svs[0] + svs[1]) + (svs[2] + svs[3]))
    cnt_gt_vec = _all_sum((cvs[0] + cvs[1]) + (cvs[2] + cvs[3]))

    ki_vec = jnp.full((16,), k_i, jnp.int32)
    tie_vec = ki_vec - cnt_gt_vec
    total_vec = sum_gt_vec + jnp.where(
        tie_vec > 0, tie_vec.astype(jnp.float32) * tval_splat,
        jnp.float32(0.0))
    return total_vec / ki_vec.astype(jnp.float32)


def _sc_body(scores_hbm, seqlen_hbm, out_hbm,
             rows_v, fbuf, seql_v, outv, sem0, sem1):
    w = lax.axis_index("s") * NC + lax.axis_index("c")
    r0 = w * ROWS_PER_W
    cp0 = pltpu.async_copy(scores_hbm.at[r0], rows_v.at[0, pl.ds(0, N)], sem0)
    cp1 = pltpu.async_copy(scores_hbm.at[r0 + 1], rows_v.at[1, pl.ds(0, N)],
                           sem1)
    pltpu.sync_copy(seqlen_hbm, seql_v.at[pl.ds(0, B)])

    seq_vec = seql_v[pl.ds(r0, 16)]
    cp0.wait()
    mean0 = _process_row(rows_v.at[0], fbuf, seq_vec[0])
    cp1.wait()
    mean1 = _process_row(rows_v.at[1], fbuf, seq_vec[1])

    iota = lax.iota(jnp.int32, 16)
    vec = jnp.where(iota == 0, mean0,
                    jnp.where(iota == 1, mean1, jnp.float32(0.0)))
    outv[...] = vec.astype(jnp.float32)
    pltpu.sync_copy(outv, out_hbm.at[w])


_sc_means = functools.partial(
    pl.kernel,
    out_type=jax.ShapeDtypeStruct((NW, 16), jnp.float32),
    mesh=plsc.VectorSubcoreMesh(core_axis_name="c", subcore_axis_name="s"),
    scratch_types=[
        pltpu.VMEM((ROWS_PER_W, N + 16), jnp.float32),
        pltpu.VMEM((N + 128,), jnp.float32),
        pltpu.VMEM((B + 16,), jnp.int32),
        pltpu.VMEM((16,), jnp.float32),
        pltpu.SemaphoreType.DMA,
        pltpu.SemaphoreType.DMA,
    ],
)(_sc_body)


def _bce_body(means_ref, label_ref, out_ref):
    m = means_ref[...][:, :ROWS_PER_W]
    lab = label_ref[...]
    p = jax.nn.sigmoid(m)
    eps = 1e-7
    p = jnp.clip(p, eps, 1.0 - eps)
    bce = -(lab * jnp.log(p) + (1.0 - lab) * jnp.log(1.0 - p))
    out_ref[0, 0] = jnp.mean(bce)


@jax.jit
def kernel(scores, label, seqlen):
    means_tile = _sc_means(scores, seqlen.astype(jnp.int32))
    out = pl.pallas_call(
        _bce_body,
        out_shape=jax.ShapeDtypeStruct((1, 1), jnp.float32),
        out_specs=pl.BlockSpec(memory_space=pltpu.SMEM),
    )(means_tile, label.reshape(NW, ROWS_PER_W))
    return out[0, 0]
